# speedup vs baseline: 1.0017x; 1.0017x over previous
import functools
import math

import jax
import jax.numpy as jnp
import numpy as np
from jax import lax
from jax.experimental import pallas as pl
from jax.experimental.pallas import tpu as pltpu
from jax.experimental.pallas import tpu_sc as plsc


def _pe_div_sqrt_d(seq_len: int, d_model: int) -> np.ndarray:
    position = np.arange(seq_len, dtype=np.float32)[:, None]
    div_term = np.exp(
        np.arange(0, d_model, 2, dtype=np.float32) * -(math.log(10000.0) / d_model)
    )
    pe = np.zeros((seq_len, d_model), dtype=np.float32)
    pe[:, 0::2] = np.sin(position * div_term)
    pe[:, 1::2] = np.cos(position * div_term)
    return pe / math.sqrt(d_model)


@functools.lru_cache(maxsize=None)
def _build(B: int, S: int, V: int, D: int):
    info = plsc.get_sparse_core_info()
    NC, NS, L = info.num_cores, info.num_subcores, info.num_lanes
    NW = NC * NS
    N = B * S
    assert N % NW == 0
    rows_per_w = N // NW
    CHUNK = 128
    n_chunks = rows_per_w // CHUNK
    assert rows_per_w % CHUNK == 0 and D % L == 0
    assert S % rows_per_w == 0

    sqrt_d = np.float32(math.sqrt(D))
    mesh = plsc.VectorSubcoreMesh(core_axis_name="c", subcore_axis_name="s")

    @functools.partial(
        pl.kernel,
        mesh=mesh,
        out_type=jax.ShapeDtypeStruct((N, D), jnp.float32),
        scratch_types=[
            pltpu.VMEM((n_chunks, CHUNK), jnp.int32),
            pltpu.VMEM((rows_per_w, D), jnp.float32),
            pltpu.SemaphoreType.DMA,
        ],
    )
    def k(x_hbm, pe_hbm, table_hbm, out_hbm, idx_v, buf, sem):
        wid = lax.axis_index("s") * NC + lax.axis_index("c")
        base = wid * rows_per_w
        pos0 = lax.rem(base, S)
        pltpu.sync_copy(x_hbm.at[pl.ds(wid * n_chunks, n_chunks)], idx_v)
        pltpu.sync_copy(pe_hbm.at[pl.ds(pos0, rows_per_w)], buf)
        copies = [
            pltpu.async_copy(
                table_hbm.at[idx_v.at[c]],
                buf.at[pl.ds(c * CHUNK, CHUNK)],
                sem,
                add=True,
            )
            for c in range(n_chunks)
        ]
        for cp in copies:
            cp.wait()

        def scale_row(r, _):
            for j in range(D // L):
                sl = pl.ds(j * L, L)
                buf[r, sl] = buf[r, sl] * sqrt_d
            return _

        lax.fori_loop(0, rows_per_w, scale_row, None)
        pltpu.sync_copy(buf, out_hbm.at[pl.ds(base, rows_per_w)])

    return k, rows_per_w, CHUNK


def kernel(x, table):
    B, S = x.shape
    V, D = table.shape
    k, rows_per_w, CHUNK = _build(B, S, V, D)
    pe = jnp.asarray(_pe_div_sqrt_d(S, D))
    x_chunked = x.reshape(-1, CHUNK).astype(jnp.int32)
    out = k(x_chunked, pe, table)
    return out.reshape(B, S, D)


# speedup vs baseline: 1.0677x; 1.0659x over previous
import functools
import math

import jax
import jax.numpy as jnp
import numpy as np
from jax import lax
from jax.experimental import pallas as pl
from jax.experimental.pallas import tpu as pltpu
from jax.experimental.pallas import tpu_sc as plsc


def _pe_div_sqrt_d(seq_len: int, d_model: int) -> np.ndarray:
    position = np.arange(seq_len, dtype=np.float32)[:, None]
    div_term = np.exp(
        np.arange(0, d_model, 2, dtype=np.float32) * -(math.log(10000.0) / d_model)
    )
    pe = np.zeros((seq_len, d_model), dtype=np.float32)
    pe[:, 0::2] = np.sin(position * div_term)
    pe[:, 1::2] = np.cos(position * div_term)
    return pe / math.sqrt(d_model)


@functools.lru_cache(maxsize=None)
def _build(B: int, S: int, V: int, D: int):
    info = plsc.get_sparse_core_info()
    NC, NS, L = info.num_cores, info.num_subcores, info.num_lanes
    NW = NC * NS
    N = B * S
    assert N % NW == 0
    rows_per_w = N // NW
    CHUNK = 128
    n_chunks = rows_per_w // CHUNK
    assert rows_per_w % CHUNK == 0 and D % L == 0
    assert S % rows_per_w == 0

    sqrt_d = np.float32(math.sqrt(D))
    mesh = plsc.VectorSubcoreMesh(core_axis_name="c", subcore_axis_name="s")

    @functools.partial(
        pl.kernel,
        mesh=mesh,
        out_type=jax.ShapeDtypeStruct((N, D), jnp.float32),
        scratch_types=[
            pltpu.VMEM((n_chunks, CHUNK), jnp.int32),
            pltpu.VMEM((rows_per_w, D), jnp.float32),
            pltpu.SemaphoreType.DMA((n_chunks,)),
            pltpu.SemaphoreType.DMA((n_chunks,)),
            pltpu.SemaphoreType.DMA((n_chunks,)),
        ],
    )
    def k(x_hbm, pe_hbm, table_hbm, out_hbm, idx_v, buf, sem_pe, sem_g, sem_o):
        wid = lax.axis_index("s") * NC + lax.axis_index("c")
        base = wid * rows_per_w
        pos0 = lax.rem(base, S)
        pltpu.sync_copy(x_hbm.at[pl.ds(wid * n_chunks, n_chunks)], idx_v)
        pe_cps = [
            pltpu.async_copy(
                pe_hbm.at[pl.ds(pos0 + c * CHUNK, CHUNK)],
                buf.at[pl.ds(c * CHUNK, CHUNK)],
                sem_pe.at[c],
            )
            for c in range(n_chunks)
        ]
        g_cps = []
        for c in range(n_chunks):
            pe_cps[c].wait()
            g_cps.append(
                pltpu.async_copy(
                    table_hbm.at[idx_v.at[c]],
                    buf.at[pl.ds(c * CHUNK, CHUNK)],
                    sem_g.at[c],
                    add=True,
                )
            )

        def scale_row(r, _):
            for j in range(D // L):
                sl = pl.ds(j * L, L)
                buf[r, sl] = buf[r, sl] * sqrt_d
            return _

        o_cps = []
        for c in range(n_chunks):
            g_cps[c].wait()
            lax.fori_loop(c * CHUNK, (c + 1) * CHUNK, scale_row, None)
            o_cps.append(
                pltpu.async_copy(
                    buf.at[pl.ds(c * CHUNK, CHUNK)],
                    out_hbm.at[pl.ds(base + c * CHUNK, CHUNK)],
                    sem_o.at[c],
                )
            )
        for cp in o_cps:
            cp.wait()

    return k, rows_per_w, CHUNK


def kernel(x, table):
    B, S = x.shape
    V, D = table.shape
    k, rows_per_w, CHUNK = _build(B, S, V, D)
    pe = jnp.asarray(_pe_div_sqrt_d(S, D))
    x_chunked = x.reshape(-1, CHUNK).astype(jnp.int32)
    out = k(x_chunked, pe, table)
    return out.reshape(B, S, D)


# speedup vs baseline: 1.1109x; 1.0405x over previous
import functools
import math

import jax
import jax.numpy as jnp
import numpy as np
from jax import lax
from jax.experimental import pallas as pl
from jax.experimental.pallas import tpu as pltpu
from jax.experimental.pallas import tpu_sc as plsc


def _pe_div_sqrt_d(seq_len: int, d_model: int) -> np.ndarray:
    position = np.arange(seq_len, dtype=np.float32)[:, None]
    div_term = np.exp(
        np.arange(0, d_model, 2, dtype=np.float32) * -(math.log(10000.0) / d_model)
    )
    pe = np.zeros((seq_len, d_model), dtype=np.float32)
    pe[:, 0::2] = np.sin(position * div_term)
    pe[:, 1::2] = np.cos(position * div_term)
    return pe / math.sqrt(d_model)


@functools.lru_cache(maxsize=None)
def _build(B: int, S: int, V: int, D: int):
    info = plsc.get_sparse_core_info()
    NC, NS, L = info.num_cores, info.num_subcores, info.num_lanes
    NW = NC * NS
    N = B * S
    assert N % NW == 0
    rows_per_w = N // NW
    CHUNK = 128
    n_chunks = rows_per_w // CHUNK
    assert rows_per_w % CHUNK == 0 and D % L == 0
    assert S % rows_per_w == 0
    slabs_per_batch = S // rows_per_w

    sqrt_d = np.float32(math.sqrt(D))
    mesh = plsc.VectorSubcoreMesh(core_axis_name="c", subcore_axis_name="s")

    @functools.partial(
        pl.kernel,
        mesh=mesh,
        out_type=jax.ShapeDtypeStruct((B, S, D), jnp.float32),
        scratch_types=[
            pltpu.VMEM((rows_per_w,), jnp.int32),
            pltpu.VMEM((rows_per_w, D), jnp.float32),
            pltpu.SemaphoreType.DMA((n_chunks,)),
            pltpu.SemaphoreType.DMA((n_chunks,)),
            pltpu.SemaphoreType.DMA((n_chunks,)),
        ],
    )
    def k(x_hbm, pe_hbm, table_hbm, out_hbm, idx_v, buf, sem_pe, sem_g, sem_o):
        wid = lax.axis_index("s") * NC + lax.axis_index("c")
        b = wid // slabs_per_batch
        off = lax.rem(wid, slabs_per_batch) * rows_per_w
        pltpu.sync_copy(x_hbm.at[b, pl.ds(off, rows_per_w)], idx_v)
        pe_cps = [
            pltpu.async_copy(
                pe_hbm.at[pl.ds(off + c * CHUNK, CHUNK)],
                buf.at[pl.ds(c * CHUNK, CHUNK)],
                sem_pe.at[c],
            )
            for c in range(n_chunks)
        ]
        g_cps = []
        for c in range(n_chunks):
            pe_cps[c].wait()
            g_cps.append(
                pltpu.async_copy(
                    table_hbm.at[idx_v.at[pl.ds(c * CHUNK, CHUNK)]],
                    buf.at[pl.ds(c * CHUNK, CHUNK)],
                    sem_g.at[c],
                    add=True,
                )
            )

        def scale_row(r, _):
            for j in range(D // L):
                sl = pl.ds(j * L, L)
                buf[r, sl] = buf[r, sl] * sqrt_d
            return _

        o_cps = []
        for c in range(n_chunks):
            g_cps[c].wait()
            lax.fori_loop(c * CHUNK, (c + 1) * CHUNK, scale_row, None)
            o_cps.append(
                pltpu.async_copy(
                    buf.at[pl.ds(c * CHUNK, CHUNK)],
                    out_hbm.at[b, pl.ds(off + c * CHUNK, CHUNK)],
                    sem_o.at[c],
                )
            )
        for cp in o_cps:
            cp.wait()

    return k


def kernel(x, table):
    B, S = x.shape
    V, D = table.shape
    k = _build(B, S, V, D)
    pe = jnp.asarray(_pe_div_sqrt_d(S, D))
    return k(x.astype(jnp.int32), pe, table)
